# Initial kernel scaffold; baseline (speedup 1.0000x reference)
import jax
import jax.numpy as jnp
from jax.experimental import pallas as pl


def kernel(h, adj, W, a):
    raise NotImplementedError("write your pallas kernel here")


# speedup vs baseline: 10.6327x; 10.6327x over previous
import jax
import jax.numpy as jnp
from jax import lax
from jax.experimental import pallas as pl
from jax.experimental.pallas import tpu as pltpu
from jax.experimental.pallas import tpu_sc as plsc

N = 10000
D = 128
E = 320000

NC = 2
NS = 16
NW = NC * NS
EPW = E // NW
GB = 80
NB = EPW // GB
NPAD = 10240
RPT = NPAD // NS

BLK = 1000


def _dense_body(h_ref, w_ref, a_ref, wh_ref, s1_ref, s2_ref, m1_ref, m2_ref):
    i = pl.program_id(0)
    wh = jnp.dot(h_ref[...], w_ref[...], preferred_element_type=jnp.float32)
    wh_ref[...] = wh
    s1 = jnp.dot(wh, a_ref[0:D, :], preferred_element_type=jnp.float32)
    s2 = jnp.dot(wh, a_ref[D:, :], preferred_element_type=jnp.float32)
    s1_ref[...] = s1
    s2_ref[...] = s2
    b1 = jnp.max(s1)
    b2 = jnp.max(s2)

    @pl.when(i == 0)
    def _():
        m1_ref[...] = jnp.full((1, 1), b1, jnp.float32)
        m2_ref[...] = jnp.full((1, 1), b2, jnp.float32)

    @pl.when(i != 0)
    def _():
        m1_ref[...] = jnp.maximum(m1_ref[...], b1)
        m2_ref[...] = jnp.maximum(m2_ref[...], b2)


def _final_body(a0_ref, a1_ref, s0_ref, s1_ref, o_ref):
    s = s0_ref[...] + s1_ref[...] + 1e-10
    x = (a0_ref[...] + a1_ref[...]) / s
    o_ref[...] = jnp.where(x > 0.0, x, jnp.exp(x) - 1.0)


def _edge_body(adj_ref, wh_ref, s1_ref, s2_ref, mv_ref, z2_ref, z1_ref,
               acc_out, sum_out, rows_b, cols_b, s1_v, s2_v, expb_v, mv_v,
               whb_v, acc_s, sum_s):
    cid = lax.axis_index("c")
    sid = lax.axis_index("s")
    wid = sid * NC + cid

    pltpu.sync_copy(z2_ref, acc_s.at[pl.ds(sid * RPT, RPT), :])
    pltpu.sync_copy(z1_ref, sum_s.at[pl.ds(sid * RPT, RPT)])

    pltpu.sync_copy(s1_ref, s1_v)
    pltpu.sync_copy(s2_ref, s2_v)
    pltpu.sync_copy(mv_ref, mv_v)
    mvec = mv_v[...]

    plsc.subcore_barrier()

    def batch(b, _):
        pltpu.sync_copy(adj_ref.at[0, wid * NB + b], rows_b)
        pltpu.sync_copy(adj_ref.at[1, wid * NB + b], cols_b)
        pltpu.sync_copy(wh_ref.at[cols_b.at[0]], whb_v)

        for j in range(GB // 16):
            sl = pl.ds(j * 16, 16)
            r16 = rows_b[0, sl]
            c16 = cols_b[0, sl]
            e = plsc.load_gather(s1_v, [r16]) + plsc.load_gather(s2_v, [c16])
            e = jnp.where(e > 0.0, e, 0.2 * e) - mvec
            w16 = jnp.exp(e)
            expb_v[0, sl] = w16
            for t in range(16):
                w = w16[t]
                g = j * 16 + t
                for d in range(D // 16):
                    dl = pl.ds(d * 16, 16)
                    whb_v[g, dl] = whb_v[g, dl] * w

        pltpu.sync_copy(whb_v, acc_s.at[rows_b.at[0]], add=True)
        pltpu.sync_copy(expb_v.at[0], sum_s.at[rows_b.at[0]], add=True)
        return 0

    lax.fori_loop(0, NB, batch, 0)

    plsc.subcore_barrier()

    pltpu.sync_copy(acc_s.at[pl.ds(sid * RPT, RPT), :],
                    acc_out.at[cid, pl.ds(sid * RPT, RPT), :])
    pltpu.sync_copy(sum_s.at[pl.ds(sid * RPT, RPT)],
                    sum_out.at[cid, pl.ds(sid * RPT, RPT)])


_edge_kernel = pl.kernel(
    _edge_body,
    out_type=(
        jax.ShapeDtypeStruct((NC, NPAD, D), jnp.float32),
        jax.ShapeDtypeStruct((NC, NPAD), jnp.float32),
    ),
    mesh=plsc.VectorSubcoreMesh(core_axis_name="c", subcore_axis_name="s"),
    scratch_types=(
        pltpu.VMEM((1, GB), jnp.int32),
        pltpu.VMEM((1, GB), jnp.int32),
        pltpu.VMEM((N,), jnp.float32),
        pltpu.VMEM((N,), jnp.float32),
        pltpu.VMEM((1, GB), jnp.float32),
        pltpu.VMEM((16,), jnp.float32),
        pltpu.VMEM((GB, D), jnp.float32),
        pltpu.VMEM_SHARED((NPAD, D), jnp.float32),
        pltpu.VMEM_SHARED((NPAD,), jnp.float32),
    ),
    compiler_params=pltpu.CompilerParams(needs_layout_passes=False),
)


@jax.jit
def kernel(h, adj, W, a):
    nblk = N // BLK
    wh, s1, s2, m1, m2 = pl.pallas_call(
        _dense_body,
        grid=(nblk,),
        in_specs=[
            pl.BlockSpec((BLK, D), lambda i: (i, 0)),
            pl.BlockSpec((D, D), lambda i: (0, 0)),
            pl.BlockSpec((2 * D, 1), lambda i: (0, 0)),
        ],
        out_specs=[
            pl.BlockSpec((BLK, D), lambda i: (i, 0)),
            pl.BlockSpec((BLK, 1), lambda i: (i, 0)),
            pl.BlockSpec((BLK, 1), lambda i: (i, 0)),
            pl.BlockSpec((1, 1), lambda i: (0, 0)),
            pl.BlockSpec((1, 1), lambda i: (0, 0)),
        ],
        out_shape=[
            jax.ShapeDtypeStruct((N, D), jnp.float32),
            jax.ShapeDtypeStruct((N, 1), jnp.float32),
            jax.ShapeDtypeStruct((N, 1), jnp.float32),
            jax.ShapeDtypeStruct((1, 1), jnp.float32),
            jax.ShapeDtypeStruct((1, 1), jnp.float32),
        ],
    )(h, W, a)

    m = m1[0, 0] + m2[0, 0]
    mshift = jnp.where(m > 0.0, m, 0.2 * m)
    mvec = jnp.full((16,), mshift, jnp.float32)
    adj_r = adj.reshape(2, NW * NB, 1, GB)
    z2 = jnp.zeros((RPT, D), jnp.float32)
    z1 = jnp.zeros((RPT,), jnp.float32)

    acc_parts, sum_parts = _edge_kernel(
        adj_r, wh, s1.reshape(N), s2.reshape(N), mvec, z2, z1)

    a0 = acc_parts[0]
    a1 = acc_parts[1]
    s0 = sum_parts[0].reshape(NPAD, 1)
    s1p = sum_parts[1].reshape(NPAD, 1)

    out = pl.pallas_call(
        _final_body,
        grid=(nblk,),
        in_specs=[
            pl.BlockSpec((BLK, D), lambda i: (i, 0)),
            pl.BlockSpec((BLK, D), lambda i: (i, 0)),
            pl.BlockSpec((BLK, 1), lambda i: (i, 0)),
            pl.BlockSpec((BLK, 1), lambda i: (i, 0)),
        ],
        out_specs=pl.BlockSpec((BLK, D), lambda i: (i, 0)),
        out_shape=jax.ShapeDtypeStruct((N, D), jnp.float32),
    )(a0, a1, s0, s1p)
    return out


# speedup vs baseline: 15.0826x; 1.4185x over previous
import jax
import jax.numpy as jnp
from jax import lax
from jax.experimental import pallas as pl
from jax.experimental.pallas import tpu as pltpu
from jax.experimental.pallas import tpu_sc as plsc

N = 10000
D = 128
E = 320000

NC = 2
NS = 16
NW = NC * NS
EPW = E // NW
GB = 80
NB = EPW // GB
NPAD = 10240
RPT = NPAD // NS

BLK = 1000


def _dense_body(h_ref, w_ref, a_ref, wh_ref, s1_ref, s2_ref, m1_ref, m2_ref):
    i = pl.program_id(0)
    wh = jnp.dot(h_ref[...], w_ref[...], preferred_element_type=jnp.float32)
    wh_ref[...] = wh
    s1 = jnp.dot(wh, a_ref[0:D, :], preferred_element_type=jnp.float32)
    s2 = jnp.dot(wh, a_ref[D:, :], preferred_element_type=jnp.float32)
    s1_ref[...] = s1
    s2_ref[...] = s2
    b1 = jnp.max(s1)
    b2 = jnp.max(s2)

    @pl.when(i == 0)
    def _():
        m1_ref[...] = jnp.full((1, 1), b1, jnp.float32)
        m2_ref[...] = jnp.full((1, 1), b2, jnp.float32)

    @pl.when(i != 0)
    def _():
        m1_ref[...] = jnp.maximum(m1_ref[...], b1)
        m2_ref[...] = jnp.maximum(m2_ref[...], b2)


def _final_body(a0_ref, a1_ref, s0_ref, s1_ref, o_ref):
    s = s0_ref[...] + s1_ref[...] + 1e-10
    x = (a0_ref[...] + a1_ref[...]) / s
    o_ref[...] = jnp.where(x > 0.0, x, jnp.exp(x) - 1.0)


def _edge_body(adj_ref, wh_ref, s1h_ref, s2h_ref, mv_ref, z2_ref, z1_ref,
               acc_out, sum_out, rows_b, cols_b, s1_b, s2_b, expb_v, mv_v,
               whb_v, acc_s, sum_s, gsem_w, gsem_s, ssem_a, ssem_s):
    cid = lax.axis_index("c")
    sid = lax.axis_index("s")
    wid = sid * NC + cid
    base = wid * NB

    pltpu.sync_copy(z2_ref, acc_s.at[pl.ds(sid * RPT, RPT), :])
    pltpu.sync_copy(z1_ref, sum_s.at[pl.ds(sid * RPT, RPT)])
    pltpu.sync_copy(mv_ref, mv_v)
    mvec = mv_v[...]

    plsc.subcore_barrier()

    def stage(k, slot):
        pltpu.sync_copy(adj_ref.at[0, base + k], rows_b.at[slot])
        pltpu.sync_copy(adj_ref.at[1, base + k], cols_b.at[slot])
        pltpu.async_copy(s1h_ref.at[rows_b.at[slot, 0]],
                         s1_b.at[slot, 0], gsem_s.at[slot])
        pltpu.async_copy(s2h_ref.at[cols_b.at[slot, 0]],
                         s2_b.at[slot, 0], gsem_s.at[slot])
        pltpu.async_copy(wh_ref.at[cols_b.at[slot, 0]],
                         whb_v.at[slot], gsem_w.at[slot])

    stage(0, 0)

    def batch(b, _):
        slot = jnp.bitwise_and(b, 1)
        nxt = 1 - slot

        @pl.when(b >= 1)
        def _():
            pltpu.make_async_copy(
                whb_v.at[nxt], acc_s.at[rows_b.at[nxt, 0]],
                ssem_a.at[nxt]).wait()
            pltpu.make_async_copy(
                expb_v.at[nxt, 0], sum_s.at[rows_b.at[nxt, 0]],
                ssem_s.at[nxt]).wait()

        @pl.when(b + 1 < NB)
        def _():
            stage(b + 1, nxt)

        pltpu.make_async_copy(s1h_ref.at[rows_b.at[slot, 0]],
                              s1_b.at[slot, 0], gsem_s.at[slot]).wait()
        pltpu.make_async_copy(s2h_ref.at[cols_b.at[slot, 0]],
                              s2_b.at[slot, 0], gsem_s.at[slot]).wait()
        pltpu.make_async_copy(wh_ref.at[cols_b.at[slot, 0]],
                              whb_v.at[slot], gsem_w.at[slot]).wait()

        for j in range(GB // 16):
            sl = pl.ds(j * 16, 16)
            e = s1_b[slot, 0, sl] + s2_b[slot, 0, sl]
            e = jnp.where(e > 0.0, e, 0.2 * e) - mvec
            w16 = jnp.exp(e)
            expb_v[slot, 0, sl] = w16
            for t in range(16):
                w = w16[t]
                g = j * 16 + t
                for d in range(D // 16):
                    dl = pl.ds(d * 16, 16)
                    whb_v[slot, g, dl] = whb_v[slot, g, dl] * w

        pltpu.async_copy(whb_v.at[slot], acc_s.at[rows_b.at[slot, 0]],
                         ssem_a.at[slot], add=True)
        pltpu.async_copy(expb_v.at[slot, 0], sum_s.at[rows_b.at[slot, 0]],
                         ssem_s.at[slot], add=True)
        return 0

    lax.fori_loop(0, NB, batch, 0)

    last = (NB - 1) & 1
    pltpu.make_async_copy(whb_v.at[last], acc_s.at[rows_b.at[last, 0]],
                          ssem_a.at[last]).wait()
    pltpu.make_async_copy(expb_v.at[last, 0], sum_s.at[rows_b.at[last, 0]],
                          ssem_s.at[last]).wait()

    plsc.subcore_barrier()

    pltpu.sync_copy(acc_s.at[pl.ds(sid * RPT, RPT), :],
                    acc_out.at[cid, pl.ds(sid * RPT, RPT), :])
    pltpu.sync_copy(sum_s.at[pl.ds(sid * RPT, RPT)],
                    sum_out.at[cid, pl.ds(sid * RPT, RPT)])


_edge_kernel = pl.kernel(
    _edge_body,
    out_type=(
        jax.ShapeDtypeStruct((NC, NPAD, D), jnp.float32),
        jax.ShapeDtypeStruct((NC, NPAD), jnp.float32),
    ),
    mesh=plsc.VectorSubcoreMesh(core_axis_name="c", subcore_axis_name="s"),
    scratch_types=(
        pltpu.VMEM((2, 1, GB), jnp.int32),
        pltpu.VMEM((2, 1, GB), jnp.int32),
        pltpu.VMEM((2, 1, GB), jnp.float32),
        pltpu.VMEM((2, 1, GB), jnp.float32),
        pltpu.VMEM((2, 1, GB), jnp.float32),
        pltpu.VMEM((16,), jnp.float32),
        pltpu.VMEM((2, GB, D), jnp.float32),
        pltpu.VMEM_SHARED((NPAD, D), jnp.float32),
        pltpu.VMEM_SHARED((NPAD,), jnp.float32),
        pltpu.SemaphoreType.DMA((2,)),
        pltpu.SemaphoreType.DMA((2,)),
        pltpu.SemaphoreType.DMA((2,)),
        pltpu.SemaphoreType.DMA((2,)),
    ),
    compiler_params=pltpu.CompilerParams(needs_layout_passes=False),
)


@jax.jit
def kernel(h, adj, W, a):
    nblk = N // BLK
    wh, s1, s2, m1, m2 = pl.pallas_call(
        _dense_body,
        grid=(nblk,),
        in_specs=[
            pl.BlockSpec((BLK, D), lambda i: (i, 0)),
            pl.BlockSpec((D, D), lambda i: (0, 0)),
            pl.BlockSpec((2 * D, 1), lambda i: (0, 0)),
        ],
        out_specs=[
            pl.BlockSpec((BLK, D), lambda i: (i, 0)),
            pl.BlockSpec((BLK, 1), lambda i: (i, 0)),
            pl.BlockSpec((BLK, 1), lambda i: (i, 0)),
            pl.BlockSpec((1, 1), lambda i: (0, 0)),
            pl.BlockSpec((1, 1), lambda i: (0, 0)),
        ],
        out_shape=[
            jax.ShapeDtypeStruct((N, D), jnp.float32),
            jax.ShapeDtypeStruct((N, 1), jnp.float32),
            jax.ShapeDtypeStruct((N, 1), jnp.float32),
            jax.ShapeDtypeStruct((1, 1), jnp.float32),
            jax.ShapeDtypeStruct((1, 1), jnp.float32),
        ],
    )(h, W, a)

    m = m1[0, 0] + m2[0, 0]
    mshift = jnp.where(m > 0.0, m, 0.2 * m)
    mvec = jnp.full((16,), mshift, jnp.float32)
    adj_r = adj.reshape(2, NW * NB, 1, GB)
    z2 = jnp.zeros((RPT, D), jnp.float32)
    z1 = jnp.zeros((RPT,), jnp.float32)

    acc_parts, sum_parts = _edge_kernel(
        adj_r, wh, s1.reshape(N), s2.reshape(N), mvec, z2, z1)

    a0 = acc_parts[0]
    a1 = acc_parts[1]
    s0 = sum_parts[0].reshape(NPAD, 1)
    s1p = sum_parts[1].reshape(NPAD, 1)

    out = pl.pallas_call(
        _final_body,
        grid=(nblk,),
        in_specs=[
            pl.BlockSpec((BLK, D), lambda i: (i, 0)),
            pl.BlockSpec((BLK, D), lambda i: (i, 0)),
            pl.BlockSpec((BLK, 1), lambda i: (i, 0)),
            pl.BlockSpec((BLK, 1), lambda i: (i, 0)),
        ],
        out_specs=pl.BlockSpec((BLK, D), lambda i: (i, 0)),
        out_shape=jax.ShapeDtypeStruct((N, D), jnp.float32),
    )(a0, a1, s0, s1p)
    return out


# speedup vs baseline: 23.8583x; 1.5818x over previous
import jax
import jax.numpy as jnp
from jax import lax
from jax.experimental import pallas as pl
from jax.experimental.pallas import tpu as pltpu
from jax.experimental.pallas import tpu_sc as plsc

N = 10000
D = 128
E = 320000

NC = 2
NS = 16
NW = NC * NS
EPW = E // NW
GB = 80
NB = EPW // GB
NPAD = 10240
RPT = NPAD // NS

BLK = 1000


def _dense_body(h_ref, w_ref, a_ref, wh_ref, s1_ref, s2_ref, m1_ref, m2_ref):
    i = pl.program_id(0)
    wh = jnp.dot(h_ref[...], w_ref[...], preferred_element_type=jnp.float32)
    wh_ref[...] = wh
    s1 = jnp.dot(wh, a_ref[0:D, :], preferred_element_type=jnp.float32)
    s2 = jnp.dot(wh, a_ref[D:, :], preferred_element_type=jnp.float32)
    s1_ref[...] = s1
    s2_ref[...] = s2
    b1 = jnp.max(s1)
    b2 = jnp.max(s2)

    @pl.when(i == 0)
    def _():
        m1_ref[...] = jnp.full((1, 1), b1, jnp.float32)
        m2_ref[...] = jnp.full((1, 1), b2, jnp.float32)

    @pl.when(i != 0)
    def _():
        m1_ref[...] = jnp.maximum(m1_ref[...], b1)
        m2_ref[...] = jnp.maximum(m2_ref[...], b2)


def _final_body(a0_ref, a1_ref, s0_ref, s1_ref, o_ref):
    s = s0_ref[...] + s1_ref[...] + 1e-10
    x = (a0_ref[...] + a1_ref[...]) / s
    o_ref[...] = jnp.where(x > 0.0, x, jnp.exp(x) - 1.0)


def _edge_body(adj_ref, wh_ref, s1h_ref, s2h_ref, mv_ref, z2_ref, z1_ref,
               acc_out, sum_out, rows_b, cols_b, s1_b, s2_b, expb_v, mv_v,
               whb_v, acc_s, sum_s, isem, gsem_w, gsem_s, ssem_a, ssem_s):
    cid = lax.axis_index("c")
    sid = lax.axis_index("s")
    wid = sid * NC + cid
    base = wid * NB

    pltpu.sync_copy(z2_ref, acc_s.at[pl.ds(sid * RPT, RPT), :])
    pltpu.sync_copy(z1_ref, sum_s.at[pl.ds(sid * RPT, RPT)])
    pltpu.sync_copy(mv_ref, mv_v)
    mvec = mv_v[...]

    plsc.subcore_barrier()

    def stage_idx(k):
        i4 = lax.rem(k, 4)
        pltpu.async_copy(adj_ref.at[0, base + k], rows_b.at[i4], isem.at[i4])
        pltpu.async_copy(adj_ref.at[1, base + k], cols_b.at[i4], isem.at[i4])

    def wait_idx(k):
        i4 = lax.rem(k, 4)
        pltpu.make_async_copy(adj_ref.at[0, base + k], rows_b.at[i4],
                              isem.at[i4]).wait()
        pltpu.make_async_copy(adj_ref.at[1, base + k], cols_b.at[i4],
                              isem.at[i4]).wait()

    def start_gathers(k):
        i4 = lax.rem(k, 4)
        w3 = lax.rem(k, 3)
        pltpu.async_copy(s1h_ref.at[rows_b.at[i4, 0]],
                         s1_b.at[i4, 0], gsem_s.at[i4])
        pltpu.async_copy(s2h_ref.at[cols_b.at[i4, 0]],
                         s2_b.at[i4, 0], gsem_s.at[i4])
        pltpu.async_copy(wh_ref.at[cols_b.at[i4, 0]],
                         whb_v.at[w3], gsem_w.at[w3])

    def wait_gathers(k):
        i4 = lax.rem(k, 4)
        w3 = lax.rem(k, 3)
        pltpu.make_async_copy(s1h_ref.at[rows_b.at[i4, 0]],
                              s1_b.at[i4, 0], gsem_s.at[i4]).wait()
        pltpu.make_async_copy(s2h_ref.at[cols_b.at[i4, 0]],
                              s2_b.at[i4, 0], gsem_s.at[i4]).wait()
        pltpu.make_async_copy(wh_ref.at[cols_b.at[i4, 0]],
                              whb_v.at[w3], gsem_w.at[w3]).wait()

    def start_scatters(k):
        i4 = lax.rem(k, 4)
        w3 = lax.rem(k, 3)
        pltpu.async_copy(whb_v.at[w3], acc_s.at[rows_b.at[i4, 0]],
                         ssem_a.at[w3], add=True)
        pltpu.async_copy(expb_v.at[i4, 0], sum_s.at[rows_b.at[i4, 0]],
                         ssem_s.at[i4], add=True)

    def drain_scatters(k):
        i4 = lax.rem(k, 4)
        w3 = lax.rem(k, 3)
        pltpu.make_async_copy(whb_v.at[w3], acc_s.at[rows_b.at[i4, 0]],
                              ssem_a.at[w3]).wait()
        pltpu.make_async_copy(expb_v.at[i4, 0], sum_s.at[rows_b.at[i4, 0]],
                              ssem_s.at[i4]).wait()

    stage_idx(0)
    stage_idx(1)
    wait_idx(0)
    start_gathers(0)

    def batch(b, _):
        @pl.when(b >= 2)
        def _():
            drain_scatters(b - 2)

        @pl.when(b + 2 < NB)
        def _():
            stage_idx(b + 2)

        @pl.when(b + 1 < NB)
        def _():
            wait_idx(b + 1)
            start_gathers(b + 1)

        wait_gathers(b)

        i4 = lax.rem(b, 4)
        w3 = lax.rem(b, 3)
        for j in range(GB // 16):
            sl = pl.ds(j * 16, 16)
            e = s1_b[i4, 0, sl] + s2_b[i4, 0, sl]
            e = jnp.where(e > 0.0, e, 0.2 * e) - mvec
            w16 = jnp.exp(e)
            expb_v[i4, 0, sl] = w16
            for t in range(16):
                w = w16[t]
                g = j * 16 + t
                for d in range(D // 16):
                    dl = pl.ds(d * 16, 16)
                    whb_v[w3, g, dl] = whb_v[w3, g, dl] * w

        start_scatters(b)
        return 0

    lax.fori_loop(0, NB, batch, 0)

    drain_scatters(NB - 2)
    drain_scatters(NB - 1)

    plsc.subcore_barrier()

    pltpu.sync_copy(acc_s.at[pl.ds(sid * RPT, RPT), :],
                    acc_out.at[cid, pl.ds(sid * RPT, RPT), :])
    pltpu.sync_copy(sum_s.at[pl.ds(sid * RPT, RPT)],
                    sum_out.at[cid, pl.ds(sid * RPT, RPT)])


_edge_kernel = pl.kernel(
    _edge_body,
    out_type=(
        jax.ShapeDtypeStruct((NC, NPAD, D), jnp.float32),
        jax.ShapeDtypeStruct((NC, NPAD), jnp.float32),
    ),
    mesh=plsc.VectorSubcoreMesh(core_axis_name="c", subcore_axis_name="s"),
    scratch_types=(
        pltpu.VMEM((4, 1, GB), jnp.int32),
        pltpu.VMEM((4, 1, GB), jnp.int32),
        pltpu.VMEM((4, 1, GB), jnp.float32),
        pltpu.VMEM((4, 1, GB), jnp.float32),
        pltpu.VMEM((4, 1, GB), jnp.float32),
        pltpu.VMEM((16,), jnp.float32),
        pltpu.VMEM((3, GB, D), jnp.float32),
        pltpu.VMEM_SHARED((NPAD, D), jnp.float32),
        pltpu.VMEM_SHARED((NPAD,), jnp.float32),
        pltpu.SemaphoreType.DMA((4,)),
        pltpu.SemaphoreType.DMA((3,)),
        pltpu.SemaphoreType.DMA((4,)),
        pltpu.SemaphoreType.DMA((3,)),
        pltpu.SemaphoreType.DMA((4,)),
    ),
    compiler_params=pltpu.CompilerParams(needs_layout_passes=False),
)


@jax.jit
def kernel(h, adj, W, a):
    nblk = N // BLK
    wh, s1, s2, m1, m2 = pl.pallas_call(
        _dense_body,
        grid=(nblk,),
        in_specs=[
            pl.BlockSpec((BLK, D), lambda i: (i, 0)),
            pl.BlockSpec((D, D), lambda i: (0, 0)),
            pl.BlockSpec((2 * D, 1), lambda i: (0, 0)),
        ],
        out_specs=[
            pl.BlockSpec((BLK, D), lambda i: (i, 0)),
            pl.BlockSpec((BLK, 1), lambda i: (i, 0)),
            pl.BlockSpec((BLK, 1), lambda i: (i, 0)),
            pl.BlockSpec((1, 1), lambda i: (0, 0)),
            pl.BlockSpec((1, 1), lambda i: (0, 0)),
        ],
        out_shape=[
            jax.ShapeDtypeStruct((N, D), jnp.float32),
            jax.ShapeDtypeStruct((N, 1), jnp.float32),
            jax.ShapeDtypeStruct((N, 1), jnp.float32),
            jax.ShapeDtypeStruct((1, 1), jnp.float32),
            jax.ShapeDtypeStruct((1, 1), jnp.float32),
        ],
    )(h, W, a)

    m = m1[0, 0] + m2[0, 0]
    mshift = jnp.where(m > 0.0, m, 0.2 * m)
    mvec = jnp.full((16,), mshift, jnp.float32)
    adj_r = adj.reshape(2, NW * NB, 1, GB)
    z2 = jnp.zeros((RPT, D), jnp.float32)
    z1 = jnp.zeros((RPT,), jnp.float32)

    acc_parts, sum_parts = _edge_kernel(
        adj_r, wh, s1.reshape(N), s2.reshape(N), mvec, z2, z1)

    a0 = acc_parts[0]
    a1 = acc_parts[1]
    s0 = sum_parts[0].reshape(NPAD, 1)
    s1p = sum_parts[1].reshape(NPAD, 1)

    out = pl.pallas_call(
        _final_body,
        grid=(nblk,),
        in_specs=[
            pl.BlockSpec((BLK, D), lambda i: (i, 0)),
            pl.BlockSpec((BLK, D), lambda i: (i, 0)),
            pl.BlockSpec((BLK, 1), lambda i: (i, 0)),
            pl.BlockSpec((BLK, 1), lambda i: (i, 0)),
        ],
        out_specs=pl.BlockSpec((BLK, D), lambda i: (i, 0)),
        out_shape=jax.ShapeDtypeStruct((N, D), jnp.float32),
    )(a0, a1, s0, s1p)
    return out
